# baseline (speedup 1.0000x reference)
import functools

import numpy as np
import jax
import jax.numpy as jnp
from jax.experimental import pallas as pl

N = 10000
NP = 10240
R = 128
NB = NP // R
K = 9
NCHUNK = NP // 128

_ROT = np.array([[0.5, -0.5, 0.7071],
                 [0.7071, 0.7071, 0.0],
                 [-0.5, 0.5, 0.7071]], dtype=np.float32)


def _umbrella_kernel(q_ref, kt_ref, qsq_ref, ksq_ref, kc_ref, rot_ref,
                     idx_ref, pn_ref):
    f32 = jnp.float32
    q = q_ref[...]
    kt = kt_ref[...]
    qsq = qsq_ref[...]
    ksq = ksq_ref[...]
    kc = kc_ref[...]
    rot = rot_ref[...]

    xy = jax.lax.dot_general(q, kt, (((1,), (0,)), ((), ())),
                             preferred_element_type=f32)
    d = jnp.sqrt(jnp.maximum(qsq - 2.0 * xy + ksq, 0.0))
    cols = jax.lax.broadcasted_iota(jnp.int32, (R, NP), 1)

    chunk_iota = jax.lax.broadcasted_iota(jnp.int32, (R, NCHUNK), 1)
    lane_iota = jax.lax.broadcasted_iota(jnp.int32, (R, 128), 1)
    cols9 = jax.lax.broadcasted_iota(jnp.int32, (R, K), 1)

    idx_cols = []
    gx_list, gy_list, gz_list = [], [], []
    phi_cols = []
    for k in range(K):
        m = jnp.min(d, axis=1, keepdims=True)
        t = jnp.where(d == m, cols, NP)
        idxk = jnp.min(t, axis=1, keepdims=True)
        if k < K - 1:
            d = jnp.where(t == idxk, jnp.inf, d)
        idx_cols.append(idxk)

        chunk = jax.lax.shift_right_logical(idxk, 7)
        lane = jnp.bitwise_and(idxk, 127)
        u = jnp.where(chunk_iota == chunk, 1.0, 0.0).astype(f32)
        row = jax.lax.dot_general(u, kc, (((1,), (0,)), ((), ())),
                                  precision=jax.lax.Precision.HIGHEST,
                                  preferred_element_type=f32)
        lsel = lane_iota == lane
        cx = jnp.sum(jnp.where(lsel, row[:, 0:128], 0.0), axis=1, keepdims=True)
        cy = jnp.sum(jnp.where(lsel, row[:, 128:256], 0.0), axis=1, keepdims=True)
        cz = jnp.sum(jnp.where(lsel, row[:, 256:384], 0.0), axis=1, keepdims=True)

        gx = cx - q[:, 0:1]
        gy = cy - q[:, 1:2]
        gz = cz - q[:, 2:3]
        gx_list.append(gx)
        gy_list.append(gy)
        gz_list.append(gz)

        g3 = jnp.concatenate([gx, gy, gz], axis=1)
        r3 = jax.lax.dot_general(g3, rot, (((1,), (0,)), ((), ())),
                                 preferred_element_type=f32)
        phi = jnp.arctan2(r3[:, 1:2], r3[:, 0:1]) / (2.0 * np.pi) + 0.5
        phi_cols.append(phi)

    idx_ref[...] = jnp.concatenate(idx_cols, axis=1)

    phis = jnp.concatenate(phi_cols, axis=1)
    ranks = []
    for k in range(K):
        pk = phis[:, k:k + 1]
        cmp = (phis < pk) | ((phis == pk) & (cols9 < k))
        ranks.append(jnp.sum(cmp.astype(jnp.int32), axis=1, keepdims=True))

    def place(vals):
        acc = jnp.zeros((R, K), f32)
        for k in range(K):
            acc = acc + jnp.where(cols9 == ranks[k], vals[k], 0.0)
        return acc

    sx, sy, sz = place(gx_list), place(gy_list), place(gz_list)
    rx = jnp.concatenate([sx[:, 1:], sx[:, :1]], axis=1)
    ry = jnp.concatenate([sy[:, 1:], sy[:, :1]], axis=1)
    rz = jnp.concatenate([sz[:, 1:], sz[:, :1]], axis=1)

    nx = sy * rz - sz * ry
    ny = sz * rx - sx * rz
    nz = sx * ry - sy * rx
    lensq = (nx * nx + nz * nz) + ny * ny
    length = jnp.sqrt(lensq)
    deg = length == 0.0
    safel = jnp.where(deg, 1.0, length)
    nan = jnp.float32(np.nan)
    ux = jnp.where(deg, nan, nx / safel)
    uy = jnp.where(deg, nan, ny / safel)
    uz = jnp.where(deg, nan, nz / safel)
    pm = jnp.where(ux[:, 0:1] > 0, 1.0, -1.0)
    ux, uy, uz = ux * pm, uy * pm, uz * pm
    areas = 0.5 * length

    valid = ~(jnp.isnan(ux) | jnp.isnan(uy) | jnp.isnan(uz))
    big = jnp.int32(1 << 30)
    fi = jnp.min(jnp.where(valid, cols9, big), axis=1, keepdims=True)
    fi = jnp.where(fi == big, 0, fi)
    fsel = cols9 == fi
    fx = jnp.sum(jnp.where(fsel, ux, 0.0), axis=1, keepdims=True)
    fy = jnp.sum(jnp.where(fsel, uy, 0.0), axis=1, keepdims=True)
    fz = jnp.sum(jnp.where(fsel, uz, 0.0), axis=1, keepdims=True)
    ux = jnp.where(valid, ux, fx)
    uy = jnp.where(valid, uy, fy)
    uz = jnp.where(valid, uz, fz)

    x = areas / 1e-4
    xm = jnp.max(x, axis=1, keepdims=True)
    e = jnp.exp(x - xm)
    s = jnp.sum(e, axis=1, keepdims=True)
    w = e / s

    pnx = jnp.sum(ux * w, axis=1, keepdims=True)
    pny = jnp.sum(uy * w, axis=1, keepdims=True)
    pnz = jnp.sum(uz * w, axis=1, keepdims=True)
    pn_ref[...] = jnp.concatenate([pnx, pny, pnz], axis=1)


def _pallas_part(q, kt, qsq, ksq, kc, rot, interpret=False):
    nq = q.shape[0]
    grid_spec = pl.GridSpec(
        grid=(nq // R,),
        in_specs=[
            pl.BlockSpec((R, 3), lambda i: (i, 0)),
            pl.BlockSpec((3, NP), lambda i: (0, 0)),
            pl.BlockSpec((R, 1), lambda i: (i, 0)),
            pl.BlockSpec((1, NP), lambda i: (0, 0)),
            pl.BlockSpec((NCHUNK, 384), lambda i: (0, 0)),
            pl.BlockSpec((3, 3), lambda i: (0, 0)),
        ],
        out_specs=[
            pl.BlockSpec((R, K), lambda i: (i, 0)),
            pl.BlockSpec((R, 3), lambda i: (i, 0)),
        ],
    )
    return pl.pallas_call(
        _umbrella_kernel,
        grid_spec=grid_spec,
        out_shape=[
            jax.ShapeDtypeStruct((nq, K), jnp.int32),
            jax.ShapeDtypeStruct((nq, 3), jnp.float32),
        ],
        interpret=interpret,
    )(q, kt, qsq, ksq, kc, rot)


@functools.partial(jax.jit, static_argnames=("interpret",))
def _run(center, interpret=False):
    cpad = jnp.pad(center, ((0, NP - N), (0, 0)))
    csq = jnp.sum(center * center, axis=-1)
    csq_pad = jnp.pad(csq, (0, NP - N))
    ksq_pad = jnp.pad(csq, (0, NP - N), constant_values=np.inf)
    kc = cpad.reshape(NCHUNK, 128, 3).transpose(0, 2, 1).reshape(NCHUNK, 384)
    rot = jnp.asarray(_ROT)
    args = (cpad, cpad.T, csq_pad.reshape(NP, 1), ksq_pad.reshape(1, NP),
            kc, rot)

    idx, pn = _pallas_part(*args, interpret=interpret)
    return idx[:N], pn[:N]


def kernel(center, offset):
    del offset
    return _run(center)


# speedup vs baseline: 1.0884x; 1.0884x over previous
import functools

import numpy as np
import jax
import jax.numpy as jnp
from jax.experimental import pallas as pl

N = 10000
NP = 10240
R = 256
NB = NP // R
K = 9
NCHUNK = NP // 128

_ROT = np.array([[0.5, -0.5, 0.7071],
                 [0.7071, 0.7071, 0.0],
                 [-0.5, 0.5, 0.7071]], dtype=np.float32)


def _umbrella_kernel(q_ref, kt_ref, qsq_ref, ksq_ref, kc_ref, rot_ref,
                     idx_ref, pn_ref):
    f32 = jnp.float32
    q = q_ref[...]
    kt = kt_ref[...]
    qsq = qsq_ref[...]
    ksq = ksq_ref[...]
    kc = kc_ref[...]
    rot = rot_ref[...]

    xy = jax.lax.dot_general(q, kt, (((1,), (0,)), ((), ())),
                             preferred_element_type=f32)
    d = jnp.sqrt(jnp.maximum(qsq - 2.0 * xy + ksq, 0.0))
    cols = jax.lax.broadcasted_iota(jnp.int32, (R, NP), 1)

    chunk_iota = jax.lax.broadcasted_iota(jnp.int32, (R, NCHUNK), 1)
    lane_iota = jax.lax.broadcasted_iota(jnp.int32, (R, 128), 1)
    cols9 = jax.lax.broadcasted_iota(jnp.int32, (R, K), 1)

    idx_cols = []
    gx_list, gy_list, gz_list = [], [], []
    phi_cols = []
    for k in range(K):
        m = jnp.min(d, axis=1, keepdims=True)
        t = jnp.where(d == m, cols, NP)
        idxk = jnp.min(t, axis=1, keepdims=True)
        if k < K - 1:
            d = jnp.where(t == idxk, jnp.inf, d)
        idx_cols.append(idxk)

        chunk = jax.lax.shift_right_logical(idxk, 7)
        lane = jnp.bitwise_and(idxk, 127)
        u = jnp.where(chunk_iota == chunk, 1.0, 0.0).astype(f32)
        row = jax.lax.dot_general(u, kc, (((1,), (0,)), ((), ())),
                                  precision=jax.lax.Precision.HIGHEST,
                                  preferred_element_type=f32)
        lsel = lane_iota == lane
        cx = jnp.sum(jnp.where(lsel, row[:, 0:128], 0.0), axis=1, keepdims=True)
        cy = jnp.sum(jnp.where(lsel, row[:, 128:256], 0.0), axis=1, keepdims=True)
        cz = jnp.sum(jnp.where(lsel, row[:, 256:384], 0.0), axis=1, keepdims=True)

        gx = cx - q[:, 0:1]
        gy = cy - q[:, 1:2]
        gz = cz - q[:, 2:3]
        gx_list.append(gx)
        gy_list.append(gy)
        gz_list.append(gz)

        g3 = jnp.concatenate([gx, gy, gz], axis=1)
        r3 = jax.lax.dot_general(g3, rot, (((1,), (0,)), ((), ())),
                                 preferred_element_type=f32)
        phi = jnp.arctan2(r3[:, 1:2], r3[:, 0:1]) / (2.0 * np.pi) + 0.5
        phi_cols.append(phi)

    idx_ref[...] = jnp.concatenate(idx_cols, axis=1)

    phis = jnp.concatenate(phi_cols, axis=1)
    ranks = []
    for k in range(K):
        pk = phis[:, k:k + 1]
        cmp = (phis < pk) | ((phis == pk) & (cols9 < k))
        ranks.append(jnp.sum(cmp.astype(jnp.int32), axis=1, keepdims=True))

    def place(vals):
        acc = jnp.zeros((R, K), f32)
        for k in range(K):
            acc = acc + jnp.where(cols9 == ranks[k], vals[k], 0.0)
        return acc

    sx, sy, sz = place(gx_list), place(gy_list), place(gz_list)
    rx = jnp.concatenate([sx[:, 1:], sx[:, :1]], axis=1)
    ry = jnp.concatenate([sy[:, 1:], sy[:, :1]], axis=1)
    rz = jnp.concatenate([sz[:, 1:], sz[:, :1]], axis=1)

    nx = sy * rz - sz * ry
    ny = sz * rx - sx * rz
    nz = sx * ry - sy * rx
    lensq = (nx * nx + nz * nz) + ny * ny
    length = jnp.sqrt(lensq)
    deg = length == 0.0
    safel = jnp.where(deg, 1.0, length)
    nan = jnp.float32(np.nan)
    ux = jnp.where(deg, nan, nx / safel)
    uy = jnp.where(deg, nan, ny / safel)
    uz = jnp.where(deg, nan, nz / safel)
    pm = jnp.where(ux[:, 0:1] > 0, 1.0, -1.0)
    ux, uy, uz = ux * pm, uy * pm, uz * pm
    areas = 0.5 * length

    valid = ~(jnp.isnan(ux) | jnp.isnan(uy) | jnp.isnan(uz))
    big = jnp.int32(1 << 30)
    fi = jnp.min(jnp.where(valid, cols9, big), axis=1, keepdims=True)
    fi = jnp.where(fi == big, 0, fi)
    fsel = cols9 == fi
    fx = jnp.sum(jnp.where(fsel, ux, 0.0), axis=1, keepdims=True)
    fy = jnp.sum(jnp.where(fsel, uy, 0.0), axis=1, keepdims=True)
    fz = jnp.sum(jnp.where(fsel, uz, 0.0), axis=1, keepdims=True)
    ux = jnp.where(valid, ux, fx)
    uy = jnp.where(valid, uy, fy)
    uz = jnp.where(valid, uz, fz)

    x = areas / 1e-4
    xm = jnp.max(x, axis=1, keepdims=True)
    e = jnp.exp(x - xm)
    s = jnp.sum(e, axis=1, keepdims=True)
    w = e / s

    pnx = jnp.sum(ux * w, axis=1, keepdims=True)
    pny = jnp.sum(uy * w, axis=1, keepdims=True)
    pnz = jnp.sum(uz * w, axis=1, keepdims=True)
    pn_ref[...] = jnp.concatenate([pnx, pny, pnz], axis=1)


def _pallas_part(q, kt, qsq, ksq, kc, rot, interpret=False):
    nq = q.shape[0]
    grid_spec = pl.GridSpec(
        grid=(nq // R,),
        in_specs=[
            pl.BlockSpec((R, 3), lambda i: (i, 0)),
            pl.BlockSpec((3, NP), lambda i: (0, 0)),
            pl.BlockSpec((R, 1), lambda i: (i, 0)),
            pl.BlockSpec((1, NP), lambda i: (0, 0)),
            pl.BlockSpec((NCHUNK, 384), lambda i: (0, 0)),
            pl.BlockSpec((3, 3), lambda i: (0, 0)),
        ],
        out_specs=[
            pl.BlockSpec((R, K), lambda i: (i, 0)),
            pl.BlockSpec((R, 3), lambda i: (i, 0)),
        ],
    )
    return pl.pallas_call(
        _umbrella_kernel,
        grid_spec=grid_spec,
        out_shape=[
            jax.ShapeDtypeStruct((nq, K), jnp.int32),
            jax.ShapeDtypeStruct((nq, 3), jnp.float32),
        ],
        interpret=interpret,
    )(q, kt, qsq, ksq, kc, rot)


@functools.partial(jax.jit, static_argnames=("interpret",))
def _run(center, interpret=False):
    cpad = jnp.pad(center, ((0, NP - N), (0, 0)))
    csq = jnp.sum(center * center, axis=-1)
    csq_pad = jnp.pad(csq, (0, NP - N))
    ksq_pad = jnp.pad(csq, (0, NP - N), constant_values=np.inf)
    kc = cpad.reshape(NCHUNK, 128, 3).transpose(0, 2, 1).reshape(NCHUNK, 384)
    rot = jnp.asarray(_ROT)
    args = (cpad, cpad.T, csq_pad.reshape(NP, 1), ksq_pad.reshape(1, NP),
            kc, rot)

    idx, pn = _pallas_part(*args, interpret=interpret)
    return idx[:N], pn[:N]


def kernel(center, offset):
    del offset
    return _run(center)
